# baseline (speedup 1.0000x reference)
import jax
import jax.numpy as jnp
from jax import lax
from jax.experimental import pallas as pl
from jax.experimental.pallas import tpu as pltpu
from jax.experimental.pallas import tpu_sc as plsc

VOC = 1000000
D = 64
B = 16384
S = 200

NC = 2
NS = 16
NW = NC * NS
BPW = B // NW
C = 128
NCH = BPW // C

_LANES = 16
_NACC = D // _LANES
_CG = C // _LANES


XW = C * S // 128


def _cbow_kernel(x_hbm, table_hbm, out_hbm, xblk, idxT, acc, sem):
    wid = lax.axis_index("s") * NC + lax.axis_index("c")
    row0 = wid * BPW
    scale = jnp.float32(1.0 / S)
    lanes = lax.iota(jnp.int32, _LANES)

    def chunk(g, carry):
        base = row0 + g * C
        pltpu.sync_copy(x_hbm.at[pl.ds(base * S // 128, XW), :], xblk)

        def transpose_step(s2, c2):
            for j in range(_CG):
                flat = (lanes + (j * _LANES)) * S + s2
                v = plsc.load_gather(
                    xblk, [lax.shift_right_logical(flat, 7),
                           lax.bitwise_and(flat, 127)])
                idxT[s2, pl.ds(j * _LANES, _LANES)] = v
            return c2

        lax.fori_loop(0, S, transpose_step, 0)

        def zero_step(i, c2):
            for c in range(_NACC):
                acc[i, pl.ds(c * _LANES, _LANES)] = jnp.zeros(
                    (_LANES,), jnp.float32)
            return c2

        lax.fori_loop(0, C, zero_step, 0)

        def fire(s2, c2):
            pltpu.async_copy(table_hbm.at[idxT.at[s2]], acc, sem, add=True)
            return c2

        lax.fori_loop(0, S, fire, 0)

        def drain(s2, c2):
            pltpu.make_async_copy(table_hbm.at[idxT.at[0]], acc, sem).wait()
            return c2

        lax.fori_loop(0, S, drain, 0)

        def scale_step(i, c2):
            for c in range(_NACC):
                sl = pl.ds(c * _LANES, _LANES)
                acc[i, sl] = acc[i, sl] * scale
            return c2

        lax.fori_loop(0, C, scale_step, 0)
        pltpu.sync_copy(acc, out_hbm.at[pl.ds(base, C), :])
        return carry

    lax.fori_loop(0, NCH, chunk, 0)


@jax.jit
def _cbow(x, table):
    mesh = plsc.VectorSubcoreMesh(
        core_axis_name="c", subcore_axis_name="s",
        num_cores=NC, num_subcores=NS)
    run = pl.kernel(
        _cbow_kernel,
        out_type=jax.ShapeDtypeStruct((B, D), jnp.float32),
        mesh=mesh,
        scratch_types=[
            pltpu.VMEM((XW, 128), jnp.int32),
            pltpu.VMEM((S, C), jnp.int32),
            pltpu.VMEM((C, D), jnp.float32),
            pltpu.SemaphoreType.DMA,
        ],
        compiler_params=pltpu.CompilerParams(
            use_tc_tiling_on_sc=False, needs_layout_passes=False),
    )
    return run(x, table)


def kernel(x, word_pos, x_char, unused, table):
    del word_pos, x_char, unused
    x128 = x.astype(jnp.int32).reshape(B * S // 128, 128)
    return _cbow(x128, table)


# speedup vs baseline: 1.0467x; 1.0467x over previous
import jax
import jax.numpy as jnp
from jax import lax
from jax.experimental import pallas as pl
from jax.experimental.pallas import tpu as pltpu
from jax.experimental.pallas import tpu_sc as plsc

VOC = 1000000
D = 64
B = 16384
S = 200

NC = 2
NS = 16
NW = NC * NS
BPW = B // NW
C = 128
NCH = BPW // C
NB = B // C

_LANES = 16
_NACC = D // _LANES
SG = S // 8


def _prep_kernel(x_ref, o_ref):
    o_ref[...] = x_ref[...].T.reshape(SG, 1, 8, C)


def _cbow_kernel(idx_hbm, table_hbm, out_hbm, idxblk, acc, sem, lsem):
    wid = lax.axis_index("s") * NC + lax.axis_index("c")
    row0 = wid * BPW
    scale = jnp.float32(1.0 / S)

    def chunk(g, carry):
        base = row0 + g * C
        cb = base // C

        pltpu.async_copy(idx_hbm.at[:, cb], idxblk, lsem)

        def zero_step(i, c2):
            for c in range(_NACC):
                acc[i, pl.ds(c * _LANES, _LANES)] = jnp.zeros(
                    (_LANES,), jnp.float32)
            return c2

        lax.fori_loop(0, C, zero_step, 0)

        pltpu.make_async_copy(idx_hbm.at[:, cb], idxblk, lsem).wait()

        def fire(t, c2):
            for r in range(8):
                pltpu.async_copy(
                    table_hbm.at[plsc.Indices(idxblk.at[t, r]),
                                 pl.ds(0, D)],
                    acc, sem, add=True)
            return c2

        lax.fori_loop(0, SG, fire, 0)

        def drain(s2, c2):
            pltpu.make_async_copy(
                table_hbm.at[plsc.Indices(idxblk.at[0, 0]), pl.ds(0, D)],
                acc, sem).wait()
            return c2

        lax.fori_loop(0, S, drain, 0)

        def scale_step(i, c2):
            for c in range(_NACC):
                sl = pl.ds(c * _LANES, _LANES)
                acc[i, sl] = acc[i, sl] * scale
            return c2

        lax.fori_loop(0, C, scale_step, 0)
        pltpu.sync_copy(acc, out_hbm.at[pl.ds(base, C), :])
        return carry

    lax.fori_loop(0, NCH, chunk, 0)


@jax.jit
def _cbow(x, table):
    idx = pl.pallas_call(
        _prep_kernel,
        grid=(NB,),
        in_specs=[pl.BlockSpec((C, S), lambda i: (i, 0))],
        out_specs=pl.BlockSpec((SG, 1, 8, C), lambda i: (0, i, 0, 0)),
        out_shape=jax.ShapeDtypeStruct((SG, NB, 8, C), jnp.int32),
    )(x)

    mesh = plsc.VectorSubcoreMesh(
        core_axis_name="c", subcore_axis_name="s",
        num_cores=NC, num_subcores=NS)
    run = pl.kernel(
        _cbow_kernel,
        out_type=jax.ShapeDtypeStruct((B, D), jnp.float32),
        mesh=mesh,
        scratch_types=[
            pltpu.VMEM((SG, 8, C), jnp.int32),
            pltpu.VMEM((C, D), jnp.float32),
            pltpu.SemaphoreType.DMA,
            pltpu.SemaphoreType.DMA,
        ],
        compiler_params=pltpu.CompilerParams(
            use_tc_tiling_on_sc=False),
    )
    return run(idx, table)


def kernel(x, word_pos, x_char, unused, table):
    del word_pos, x_char, unused
    xi = x.astype(jnp.int32)
    return _cbow(xi, table)
